# baseline (speedup 1.0000x reference)
import functools
import math

import jax
import jax.numpy as jnp
from jax import lax
from jax.experimental import pallas as pl
from jax.experimental.pallas import tpu as pltpu
from jax.experimental.pallas import tpu_sc as plsc

_DECAY = 0.1
_EPS = 1e-8
_LN_EPS = 1e-5
_C = 8
_NW = 32
_VL = 16


def kernel(H, valid_lens, ln_scale, ln_bias):
    B, F, T, D = H.shape
    S = B * F
    SPW = S // _NW
    ND = D // _VL
    w0 = float(math.exp(-_DECAY * (T - 1)))
    r = float(math.exp(_DECAY))

    lens_flat = valid_lens.astype(jnp.int32).reshape(S)

    mesh = plsc.VectorSubcoreMesh(core_axis_name="c", subcore_axis_name="s")

    @functools.partial(
        pl.kernel,
        mesh=mesh,
        out_type=jax.ShapeDtypeStruct((B, F, D), jnp.float32),
        scratch_types=[
            pltpu.VMEM((S,), jnp.int32),
            pltpu.VMEM((2, (T + _C - 1) // _C, _C, D), jnp.float32),
            pltpu.VMEM((D,), jnp.float32),
            pltpu.VMEM((D,), jnp.float32),
            pltpu.VMEM((D,), jnp.float32),
            pltpu.SemaphoreType.DMA,
        ],
    )
    def sc_kernel(
        h_hbm, lens_hbm, scale_hbm, bias_hbm, out_hbm,
        lens_v, rows_v, scale_v, bias_v, out_v, dsem,
    ):
        wid = lax.axis_index("s") * 2 + lax.axis_index("c")
        pltpu.sync_copy(lens_hbm, lens_v)
        pltpu.sync_copy(scale_hbm, scale_v)
        pltpu.sync_copy(bias_hbm, bias_v)

        def chunk_copy(seg, j, slot):
            b = seg // F
            f = lax.rem(seg, F)
            return pltpu.make_async_copy(
                h_hbm.at[b, f, pl.ds(j * _C, _C), :],
                rows_v.at[slot, j],
                dsem,
            )

        def issue(i, slot):
            seg = wid * SPW + i
            ln = lens_v[pl.ds(seg, 1)][0]
            nch = (ln + (_C - 1)) // _C

            def body(j, carry):
                chunk_copy(seg, j, slot).start()
                return carry

            lax.fori_loop(0, nch, body, 0)
            return nch

        def drain(i, slot, nch):
            seg = wid * SPW + i

            def body(j, carry):
                chunk_copy(seg, j, slot).wait()
                return carry

            lax.fori_loop(0, nch, body, 0)

        def compute(i, slot):
            seg = wid * SPW + i
            ln = lens_v[pl.ds(seg, 1)][0]
            r_vec = jnp.full((_VL,), r, jnp.float32)
            w0_vec = jnp.full((_VL,), w0, jnp.float32)
            zero = jnp.zeros((_VL,), jnp.float32)
            inv_wsum = zero
            for g in range(ND // 16):
                def rbody(t, carry):
                    accs, w = carry
                    cj = t // _C
                    cr = t - cj * _C
                    new = []
                    for d in range(16):
                        col = (g * 16 + d) * _VL
                        v = rows_v[slot, cj, cr, pl.ds(col, _VL)]
                        new.append(accs[d] + v * w)
                    return (tuple(new), w * r_vec)

                accs, w_end = lax.fori_loop(
                    0, ln, rbody, (tuple([zero] * 16), w0_vec)
                )
                if g == 0:
                    wsum = (w_end - w0_vec) * (1.0 / (r - 1.0))
                    wsum = jnp.maximum(wsum, _EPS)
                    inv_wsum = 1.0 / wsum
                for d in range(16):
                    col = (g * 16 + d) * _VL
                    out_v[pl.ds(col, _VL)] = accs[d] * inv_wsum
            sm = zero
            sq = zero
            for d in range(ND):
                v = out_v[pl.ds(d * _VL, _VL)]
                sm = sm + v
                sq = sq + v * v
            sm_s = sm[0]
            sq_s = sq[0]
            for l in range(1, _VL):
                sm_s = sm_s + sm[l]
                sq_s = sq_s + sq[l]
            mu_s = sm_s * (1.0 / D)
            ex2_s = sq_s * (1.0 / D)
            mu = jnp.broadcast_to(mu_s, (_VL,))
            var = jnp.broadcast_to(ex2_s, (_VL,)) - mu * mu
            x = var + _LN_EPS
            xi = lax.bitcast_convert_type(x, jnp.int32)
            yi = jnp.int32(0x5F3759DF) - (xi >> 1)
            y = lax.bitcast_convert_type(yi, jnp.float32)
            for _ in range(4):
                y = y * (1.5 - 0.5 * x * y * y)
            m_s = jnp.where(ln >= 1, jnp.float32(1.0), jnp.float32(0.0))
            mvec = jnp.broadcast_to(m_s, (_VL,))
            for d in range(ND):
                ds_ = pl.ds(d * _VL, _VL)
                v = out_v[ds_]
                vn = (v - mu) * y * scale_v[ds_] + bias_v[ds_]
                out_v[ds_] = v + (vn - v) * mvec
            b = seg // F
            f = lax.rem(seg, F)
            pltpu.sync_copy(out_v, out_hbm.at[b, f, :])

        nch0 = issue(0, 0)

        def main(i, nch_cur):
            slot = lax.rem(i, 2)
            nch_nxt = lax.cond(
                i + 1 < SPW,
                lambda: issue(i + 1, 1 - slot),
                lambda: 0,
            )
            drain(i, slot, nch_cur)
            compute(i, slot)
            return nch_nxt

        lax.fori_loop(0, SPW, main, nch0)

    return sc_kernel(H, lens_flat, ln_scale, ln_bias)
